# baseline (speedup 1.0000x reference)
import functools

import jax
import jax.numpy as jnp
from jax import lax
from jax.experimental import pallas as pl
from jax.experimental.pallas import tpu as pltpu
from jax.experimental.pallas import tpu_sc as plsc

MAX_DEPTH = 7
RES = 2 ** MAX_DEPTH
FEATURES_DIM = 32
TABLE_SIZE = 2 ** (3 * MAX_DEPTH)
N_POS = 16384
ROWS4 = TABLE_SIZE // 4

NUM_CORES = 2
NUM_SUBCORES = 16
NUM_WORKERS = NUM_CORES * NUM_SUBCORES
CHUNK = N_POS // NUM_WORKERS
IDX_MINOR = 128
NUM_GROUPS = CHUNK // IDX_MINOR


@functools.cache
def _make_sc_gather():
    mesh = plsc.VectorSubcoreMesh(
        core_axis_name="c", subcore_axis_name="s",
        num_cores=NUM_CORES, num_subcores=NUM_SUBCORES,
    )

    @functools.partial(
        pl.kernel,
        out_type=(
            jax.ShapeDtypeStruct((N_POS, 128), jnp.float32),
            jax.ShapeDtypeStruct((N_POS,), jnp.int32),
        ),
        mesh=mesh,
        scratch_types=[
            pltpu.VMEM((CHUNK * 3,), jnp.float32),
            pltpu.VMEM((NUM_GROUPS, IDX_MINOR), jnp.int32),
            pltpu.VMEM((CHUNK,), jnp.int32),
            pltpu.VMEM((CHUNK, 128), jnp.float32),
            pltpu.SemaphoreType.DMA,
        ],
        compiler_params=pltpu.CompilerParams(
            needs_layout_passes=False,
        ),
    )
    def _sc_gather(pos_hbm, table_hbm, out_hbm, off_hbm,
                   pos_v, idx_v, off_v, rows_v, sem):
        wid = lax.axis_index("s") * NUM_CORES + lax.axis_index("c")
        base = wid * CHUNK
        pltpu.sync_copy(pos_hbm.at[pl.ds(base * 3, CHUNK * 3)], pos_v)

        lane = lax.iota(jnp.int32, 16) * 3
        for g in range(NUM_GROUPS):
            for i in range(IDX_MINOR // 16):
                off = (g * IDX_MINOR + i * 16) * 3
                x = plsc.load_gather(pos_v, [lane + off])
                y = plsc.load_gather(pos_v, [lane + (off + 1)])
                z = plsc.load_gather(pos_v, [lane + (off + 2)])
                xi = jnp.clip((x * RES).astype(jnp.int32), 0, RES - 1)
                yi = jnp.clip((y * RES).astype(jnp.int32), 0, RES - 1)
                zi = jnp.clip((z * RES).astype(jnp.int32), 0, RES - 1)
                flat = xi * (RES * RES) + yi * RES + zi
                idx_v[g, pl.ds(i * 16, 16)] = flat >> 2
                off_v[pl.ds(g * IDX_MINOR + i * 16, 16)] = flat & 3

        copies = [
            pltpu.async_copy(
                table_hbm.at[idx_v.at[g]],
                rows_v.at[pl.ds(g * IDX_MINOR, IDX_MINOR)],
                sem,
            )
            for g in range(NUM_GROUPS)
        ]
        for c in copies:
            c.wait()

        pltpu.sync_copy(rows_v, out_hbm.at[pl.ds(base, CHUNK)])
        pltpu.sync_copy(off_v, off_hbm.at[pl.ds(base, CHUNK)])

    return _sc_gather


_BM = 2048


def _mlp_body(x_ref, off_ref, w1t_ref, b1_ref, w2t_ref, b2_ref,
              rgb_ref, den_ref):
    xb = x_ref[...]
    off = off_ref[...]
    x01 = jnp.where(off < 1, xb[:, 0:32], xb[:, 32:64])
    x23 = jnp.where(off < 3, xb[:, 64:96], xb[:, 96:128])
    x = jnp.where(off < 2, x01, x23)
    h = jnp.dot(x, w1t_ref[...], preferred_element_type=jnp.float32)
    h = jnp.maximum(h + b1_ref[...], 0.0)
    o = jnp.dot(h, w2t_ref[...], preferred_element_type=jnp.float32)
    o = o + b2_ref[...]
    rgb = o[:, :3]
    den = o[:, 3:4]
    rgb_ref[...] = jnp.where(
        rgb >= 0.0,
        1.0 / (1.0 + jnp.exp(-rgb)),
        jnp.exp(rgb) / (1.0 + jnp.exp(rgb)),
    )
    den_ref[...] = jnp.maximum(den, 0.0) + jnp.log1p(jnp.exp(-jnp.abs(den)))


_mlp = pl.pallas_call(
    _mlp_body,
    grid=(N_POS // _BM,),
    in_specs=[
        pl.BlockSpec((_BM, 128), lambda i: (i, 0)),
        pl.BlockSpec((_BM, 1), lambda i: (i, 0)),
        pl.BlockSpec((FEATURES_DIM, 64), lambda i: (0, 0)),
        pl.BlockSpec((1, 64), lambda i: (0, 0)),
        pl.BlockSpec((64, 4), lambda i: (0, 0)),
        pl.BlockSpec((1, 4), lambda i: (0, 0)),
    ],
    out_specs=[
        pl.BlockSpec((_BM, 3), lambda i: (i, 0)),
        pl.BlockSpec((_BM, 1), lambda i: (i, 0)),
    ],
    out_shape=[
        jax.ShapeDtypeStruct((N_POS, 3), jnp.float32),
        jax.ShapeDtypeStruct((N_POS, 1), jnp.float32),
    ],
)


@jax.jit
def _impl(positions, octree_features, W1, b1, W2, b2):
    pos_flat = positions.reshape(-1)
    table4 = octree_features.reshape(ROWS4, 128)
    feats_big, off = _make_sc_gather()(pos_flat, table4)
    rgb, den = _mlp(
        feats_big, off.reshape(N_POS, 1),
        W1.T, b1.reshape(1, 64), W2.T, b2.reshape(1, 4),
    )
    return rgb, den


def kernel(positions, octree_features, W1, b1, W2, b2):
    return _impl(positions, octree_features, W1, b1, W2, b2)


# speedup vs baseline: 1.4385x; 1.4385x over previous
import functools

import jax
import jax.numpy as jnp
from jax import lax
from jax.experimental import pallas as pl
from jax.experimental.pallas import tpu as pltpu
from jax.experimental.pallas import tpu_sc as plsc

MAX_DEPTH = 7
RES = 2 ** MAX_DEPTH
FEATURES_DIM = 32
TABLE_SIZE = 2 ** (3 * MAX_DEPTH)
N_POS = 16384

NUM_CORES = 2
NUM_SUBCORES = 16
NUM_WORKERS = NUM_CORES * NUM_SUBCORES
CHUNK = N_POS // NUM_WORKERS
NGRP = CHUNK // 16


@functools.cache
def _make_sc_gather():
    mesh = plsc.VectorSubcoreMesh(
        core_axis_name="c", subcore_axis_name="s",
        num_cores=NUM_CORES, num_subcores=NUM_SUBCORES,
    )

    @functools.partial(
        pl.kernel,
        out_type=jax.ShapeDtypeStruct((N_POS, FEATURES_DIM), jnp.float32),
        mesh=mesh,
        scratch_types=[
            pltpu.VMEM((CHUNK * 3,), jnp.float32),
            pltpu.VMEM_SHARED((NUM_SUBCORES, CHUNK * 3), jnp.float32),
            pltpu.SMEM((CHUNK * 3,), jnp.float32),
            pltpu.VMEM((CHUNK, FEATURES_DIM), jnp.float32),
            pltpu.SemaphoreType.DMA,
            pltpu.SemaphoreType.DMA,
            pltpu.SemaphoreType.DMA,
            pltpu.SemaphoreType.DMA,
            pltpu.SemaphoreType.DMA,
        ],
        compiler_params=pltpu.CompilerParams(
            needs_layout_passes=False,
        ),
    )
    def _sc_gather(pos_hbm, table_hbm, out_hbm,
                   pos_v, pos_sh, pos_s, rows_v,
                   sem_a, sem_b, sem_c, sem_d, sem_e):
        sid = lax.axis_index("s")
        wid = sid * NUM_CORES + lax.axis_index("c")
        base = wid * CHUNK

        pltpu.async_copy(
            pos_hbm.at[pl.ds(base * 3, CHUNK * 3)], pos_v, sem_a
        ).wait()
        pl.delay(2000)
        pltpu.async_copy(pos_v, pos_sh.at[sid], sem_b).wait()
        pl.delay(2000)
        pltpu.async_copy(pos_sh.at[sid], pos_s, sem_c).wait()
        pl.delay(2000)

        def trunc_idx(v):
            r = v.astype(jnp.int32)
            r = jnp.where(r.astype(jnp.float32) > v, r - 1, r)
            return jnp.clip(r, 0, RES - 1)

        def body(i, _):
            x = pos_s[3 * i]
            y = pos_s[3 * i + 1]
            z = pos_s[3 * i + 2]
            xi = trunc_idx(x * RES)
            yi = trunc_idx(y * RES)
            zi = trunc_idx(z * RES)
            flat = xi * (RES * RES) + yi * RES + zi
            pltpu.async_copy(
                table_hbm.at[pl.ds(flat, 1)],
                rows_v.at[pl.ds(i, 1)],
                sem_d,
            )
            return _

        lax.fori_loop(0, CHUNK, body, None)

        def drain(i, _):
            pltpu.make_async_copy(
                table_hbm.at[pl.ds(0, 1)], rows_v.at[pl.ds(0, 1)], sem_d
            ).wait()
            return _

        lax.fori_loop(0, CHUNK, drain, None)

        pltpu.async_copy(
            rows_v, out_hbm.at[pl.ds(base, CHUNK)], sem_e
        ).wait()

    return _sc_gather


_BM = 2048


def _mlp_body(x_ref, w1t_ref, b1_ref, w2t_ref, b2_ref, rgb_ref, den_ref):
    x = x_ref[...]
    h = jnp.dot(x, w1t_ref[...], preferred_element_type=jnp.float32)
    h = jnp.maximum(h + b1_ref[...], 0.0)
    o = jnp.dot(h, w2t_ref[...], preferred_element_type=jnp.float32)
    o = o + b2_ref[...]
    rgb = o[:, :3]
    den = o[:, 3:4]
    rgb_ref[...] = jnp.where(
        rgb >= 0.0,
        1.0 / (1.0 + jnp.exp(-rgb)),
        jnp.exp(rgb) / (1.0 + jnp.exp(rgb)),
    )
    den_ref[...] = jnp.maximum(den, 0.0) + jnp.log1p(jnp.exp(-jnp.abs(den)))


_mlp = pl.pallas_call(
    _mlp_body,
    grid=(N_POS // _BM,),
    in_specs=[
        pl.BlockSpec((_BM, FEATURES_DIM), lambda i: (i, 0)),
        pl.BlockSpec((FEATURES_DIM, 64), lambda i: (0, 0)),
        pl.BlockSpec((1, 64), lambda i: (0, 0)),
        pl.BlockSpec((64, 4), lambda i: (0, 0)),
        pl.BlockSpec((1, 4), lambda i: (0, 0)),
    ],
    out_specs=[
        pl.BlockSpec((_BM, 3), lambda i: (i, 0)),
        pl.BlockSpec((_BM, 1), lambda i: (i, 0)),
    ],
    out_shape=[
        jax.ShapeDtypeStruct((N_POS, 3), jnp.float32),
        jax.ShapeDtypeStruct((N_POS, 1), jnp.float32),
    ],
)


@jax.jit
def _impl(positions, octree_features, W1, b1, W2, b2):
    pos_flat = positions.reshape(-1)
    feats = _make_sc_gather()(pos_flat, octree_features)
    rgb, den = _mlp(
        feats, W1.T, b1.reshape(1, 64), W2.T, b2.reshape(1, 4)
    )
    return rgb, den


def kernel(positions, octree_features, W1, b1, W2, b2):
    return _impl(positions, octree_features, W1, b1, W2, b2)


# speedup vs baseline: 1.4389x; 1.0002x over previous
import functools

import jax
import jax.numpy as jnp
from jax import lax
from jax.experimental import pallas as pl
from jax.experimental.pallas import tpu as pltpu
from jax.experimental.pallas import tpu_sc as plsc

MAX_DEPTH = 7
RES = 2 ** MAX_DEPTH
FEATURES_DIM = 32
TABLE_SIZE = 2 ** (3 * MAX_DEPTH)
N_POS = 16384

NUM_CORES = 2
NUM_SUBCORES = 16
NUM_WORKERS = NUM_CORES * NUM_SUBCORES
CHUNK = N_POS // NUM_WORKERS
NGRP = CHUNK // 16


@functools.cache
def _make_sc_gather():
    mesh = plsc.VectorSubcoreMesh(
        core_axis_name="c", subcore_axis_name="s",
        num_cores=NUM_CORES, num_subcores=NUM_SUBCORES,
    )

    @functools.partial(
        pl.kernel,
        out_type=jax.ShapeDtypeStruct((N_POS, FEATURES_DIM), jnp.float32),
        mesh=mesh,
        scratch_types=[
            pltpu.VMEM((CHUNK * 3,), jnp.float32),
            pltpu.VMEM_SHARED((NUM_SUBCORES, CHUNK * 3), jnp.float32),
            pltpu.SMEM((CHUNK * 3,), jnp.float32),
            pltpu.VMEM((CHUNK, FEATURES_DIM), jnp.float32),
            pltpu.SemaphoreType.DMA,
            pltpu.SemaphoreType.DMA,
            pltpu.SemaphoreType.DMA,
            pltpu.SemaphoreType.DMA,
            pltpu.SemaphoreType.DMA,
        ],
        compiler_params=pltpu.CompilerParams(
            needs_layout_passes=False,
            use_tc_tiling_on_sc=True,
        ),
    )
    def _sc_gather(pos_hbm, table_hbm, out_hbm,
                   pos_v, pos_sh, pos_s, rows_v,
                   sem_a, sem_b, sem_c, sem_d, sem_e):
        sid = lax.axis_index("s")
        wid = sid * NUM_CORES + lax.axis_index("c")
        base = wid * CHUNK

        pltpu.async_copy(
            pos_hbm.at[pl.ds(base * 3, CHUNK * 3)], pos_v, sem_a
        ).wait()
        pl.delay(2000)
        pltpu.async_copy(pos_v, pos_sh.at[sid], sem_b).wait()
        pl.delay(2000)
        pltpu.async_copy(pos_sh.at[sid], pos_s, sem_c).wait()
        pl.delay(2000)

        def trunc_idx(v):
            r = v.astype(jnp.int32)
            r = jnp.where(r.astype(jnp.float32) > v, r - 1, r)
            return jnp.clip(r, 0, RES - 1)

        def body(i, _):
            x = pos_s[3 * i]
            y = pos_s[3 * i + 1]
            z = pos_s[3 * i + 2]
            xi = trunc_idx(x * RES)
            yi = trunc_idx(y * RES)
            zi = trunc_idx(z * RES)
            flat = xi * (RES * RES) + yi * RES + zi
            pltpu.async_copy(
                table_hbm.at[pl.ds(flat, 1)],
                rows_v.at[pl.ds(i, 1)],
                sem_d,
            )
            return _

        lax.fori_loop(0, CHUNK, body, None)

        def drain(i, _):
            pltpu.make_async_copy(
                table_hbm.at[pl.ds(0, 1)], rows_v.at[pl.ds(0, 1)], sem_d
            ).wait()
            return _

        lax.fori_loop(0, CHUNK, drain, None)

        pltpu.async_copy(
            rows_v, out_hbm.at[pl.ds(base, CHUNK)], sem_e
        ).wait()

    return _sc_gather


_BM = 2048


def _mlp_body(x_ref, w1t_ref, b1_ref, w2t_ref, b2_ref, rgb_ref, den_ref):
    x = x_ref[...]
    h = jnp.dot(x, w1t_ref[...], preferred_element_type=jnp.float32)
    h = jnp.maximum(h + b1_ref[...], 0.0)
    o = jnp.dot(h, w2t_ref[...], preferred_element_type=jnp.float32)
    o = o + b2_ref[...]
    rgb = o[:, :3]
    den = o[:, 3:4]
    rgb_ref[...] = jnp.where(
        rgb >= 0.0,
        1.0 / (1.0 + jnp.exp(-rgb)),
        jnp.exp(rgb) / (1.0 + jnp.exp(rgb)),
    )
    den_ref[...] = jnp.maximum(den, 0.0) + jnp.log1p(jnp.exp(-jnp.abs(den)))


_mlp = pl.pallas_call(
    _mlp_body,
    grid=(N_POS // _BM,),
    in_specs=[
        pl.BlockSpec((_BM, FEATURES_DIM), lambda i: (i, 0)),
        pl.BlockSpec((FEATURES_DIM, 64), lambda i: (0, 0)),
        pl.BlockSpec((1, 64), lambda i: (0, 0)),
        pl.BlockSpec((64, 4), lambda i: (0, 0)),
        pl.BlockSpec((1, 4), lambda i: (0, 0)),
    ],
    out_specs=[
        pl.BlockSpec((_BM, 3), lambda i: (i, 0)),
        pl.BlockSpec((_BM, 1), lambda i: (i, 0)),
    ],
    out_shape=[
        jax.ShapeDtypeStruct((N_POS, 3), jnp.float32),
        jax.ShapeDtypeStruct((N_POS, 1), jnp.float32),
    ],
)


@jax.jit
def _impl(positions, octree_features, W1, b1, W2, b2):
    pos_flat = positions.reshape(-1)
    feats = _make_sc_gather()(pos_flat, octree_features)
    rgb, den = _mlp(
        feats, W1.T, b1.reshape(1, 64), W2.T, b2.reshape(1, 4)
    )
    return rgb, den


def kernel(positions, octree_features, W1, b1, W2, b2):
    return _impl(positions, octree_features, W1, b1, W2, b2)


# speedup vs baseline: 3.6205x; 2.5162x over previous
import functools

import jax
import jax.numpy as jnp
from jax import lax
from jax.experimental import pallas as pl
from jax.experimental.pallas import tpu as pltpu
from jax.experimental.pallas import tpu_sc as plsc

MAX_DEPTH = 7
RES = 2 ** MAX_DEPTH
FEATURES_DIM = 32
TABLE_SIZE = 2 ** (3 * MAX_DEPTH)
N_POS = 16384

NUM_CORES = 2
NUM_SUBCORES = 16
NUM_WORKERS = NUM_CORES * NUM_SUBCORES
CHUNK = N_POS // NUM_WORKERS
NGRP = CHUNK // 16


@functools.cache
def _make_sc_gather():
    mesh = plsc.VectorSubcoreMesh(
        core_axis_name="c", subcore_axis_name="s",
        num_cores=NUM_CORES, num_subcores=NUM_SUBCORES,
    )

    @functools.partial(
        pl.kernel,
        out_type=jax.ShapeDtypeStruct((N_POS, FEATURES_DIM), jnp.float32),
        mesh=mesh,
        scratch_types=[
            pltpu.VMEM((CHUNK * 3,), jnp.float32),
            pltpu.VMEM_SHARED((NUM_SUBCORES, CHUNK * 3), jnp.float32),
            pltpu.SMEM((CHUNK * 3,), jnp.float32),
            pltpu.VMEM((CHUNK, FEATURES_DIM), jnp.float32),
            pltpu.SemaphoreType.DMA,
            pltpu.SemaphoreType.DMA,
            pltpu.SemaphoreType.DMA,
            pltpu.SemaphoreType.DMA,
            pltpu.SemaphoreType.DMA,
        ],
        compiler_params=pltpu.CompilerParams(
            needs_layout_passes=False,
            use_tc_tiling_on_sc=True,
        ),
    )
    def _sc_gather(pos_hbm, table_hbm, out_hbm,
                   pos_v, pos_sh, pos_s, rows_v,
                   sem_a, sem_b, sem_c, sem_d, sem_e):
        sid = lax.axis_index("s")
        wid = sid * NUM_CORES + lax.axis_index("c")
        base = wid * CHUNK

        pltpu.async_copy(
            pos_hbm.at[pl.ds(base * 3, CHUNK * 3)], pos_v, sem_a
        ).wait()
        pl.delay(2000)
        pltpu.async_copy(pos_v, pos_sh.at[sid], sem_b).wait()
        pl.delay(2000)
        pltpu.async_copy(pos_sh.at[sid], pos_s, sem_c).wait()
        pl.delay(2000)

        def trunc_idx(v):
            r = v.astype(jnp.int32)
            r = jnp.where(r.astype(jnp.float32) > v, r - 1, r)
            return jnp.clip(r, 0, RES - 1)

        def body(i, _):
            x = pos_s[3 * i]
            y = pos_s[3 * i + 1]
            z = pos_s[3 * i + 2]
            xi = trunc_idx(x * RES)
            yi = trunc_idx(y * RES)
            zi = trunc_idx(z * RES)
            flat = xi * (RES * RES) + yi * RES + zi
            pltpu.async_copy(
                table_hbm.at[flat >> 5, flat & 31],
                rows_v.at[i],
                sem_d,
            )
            return _

        lax.fori_loop(0, CHUNK, body, None)

        def drain(i, _):
            pltpu.make_async_copy(
                table_hbm.at[0, 0], rows_v.at[0], sem_d
            ).wait()
            return _

        lax.fori_loop(0, CHUNK, drain, None)

        pltpu.async_copy(
            rows_v, out_hbm.at[pl.ds(base, CHUNK)], sem_e
        ).wait()

    return _sc_gather


_BM = 2048


def _mlp_body(x_ref, w1t_ref, b1_ref, w2t_ref, b2_ref, rgb_ref, den_ref):
    x = x_ref[...]
    h = jnp.dot(x, w1t_ref[...], preferred_element_type=jnp.float32)
    h = jnp.maximum(h + b1_ref[...], 0.0)
    o = jnp.dot(h, w2t_ref[...], preferred_element_type=jnp.float32)
    o = o + b2_ref[...]
    rgb = o[:, :3]
    den = o[:, 3:4]
    rgb_ref[...] = jnp.where(
        rgb >= 0.0,
        1.0 / (1.0 + jnp.exp(-rgb)),
        jnp.exp(rgb) / (1.0 + jnp.exp(rgb)),
    )
    den_ref[...] = jnp.maximum(den, 0.0) + jnp.log1p(jnp.exp(-jnp.abs(den)))


_mlp = pl.pallas_call(
    _mlp_body,
    grid=(N_POS // _BM,),
    in_specs=[
        pl.BlockSpec((_BM, FEATURES_DIM), lambda i: (i, 0)),
        pl.BlockSpec((FEATURES_DIM, 64), lambda i: (0, 0)),
        pl.BlockSpec((1, 64), lambda i: (0, 0)),
        pl.BlockSpec((64, 4), lambda i: (0, 0)),
        pl.BlockSpec((1, 4), lambda i: (0, 0)),
    ],
    out_specs=[
        pl.BlockSpec((_BM, 3), lambda i: (i, 0)),
        pl.BlockSpec((_BM, 1), lambda i: (i, 0)),
    ],
    out_shape=[
        jax.ShapeDtypeStruct((N_POS, 3), jnp.float32),
        jax.ShapeDtypeStruct((N_POS, 1), jnp.float32),
    ],
)


@jax.jit
def _impl(positions, octree_features, W1, b1, W2, b2):
    pos_flat = positions.reshape(-1)
    table3 = octree_features.reshape(TABLE_SIZE // 32, 32, FEATURES_DIM)
    feats = _make_sc_gather()(pos_flat, table3)
    rgb, den = _mlp(
        feats, W1.T, b1.reshape(1, 64), W2.T, b2.reshape(1, 4)
    )
    return rgb, den


def kernel(positions, octree_features, W1, b1, W2, b2):
    return _impl(positions, octree_features, W1, b1, W2, b2)
